# baseline (speedup 1.0000x reference)
import functools

import jax
import jax.numpy as jnp
from jax import lax
from jax.experimental import pallas as pl
from jax.experimental.pallas import tpu as pltpu
from jax.experimental.pallas import tpu_sc as plsc

NC = 2
NS = 16
NW = NC * NS
CHUNK = 64


GS = 16


def _make_seg_sum(n_pad, d, n_chunks, with_deg):
    mesh = plsc.VectorSubcoreMesh(core_axis_name="c", subcore_axis_name="s")
    rpt = n_pad // NS
    nbuf = 5
    g_depth = nbuf - 1
    n_grp = n_chunks // GS

    out_type = [jax.ShapeDtypeStruct((NC, n_pad, d), jnp.float32)]
    if with_deg:
        out_type.append(jax.ShapeDtypeStruct((NC, n_pad, d), jnp.float32))

    scratch = [pltpu.VMEM((GS, CHUNK), jnp.int32)] * 4
    scratch += [pltpu.VMEM((CHUNK, d), jnp.float32)] * nbuf
    scratch.append(pltpu.VMEM_SHARED((n_pad, d), jnp.float32))
    scratch += [pltpu.SemaphoreType.DMA] * (nbuf + 2)

    @functools.partial(
        pl.kernel, mesh=mesh, out_type=tuple(out_type),
        scratch_types=tuple(scratch),
    )
    def seg(*refs):
        if with_deg:
            (table, srcp, dstp, z_d, ones_hbm, out_sum, out_deg,
             *rest) = refs
        else:
            (table, srcp, dstp, z_d, out_sum, *rest) = refs
        idxs = rest[0:2]
        idxd = rest[2:4]
        bufs = rest[4:4 + nbuf]
        acc = rest[4 + nbuf]
        gsem = rest[5 + nbuf:5 + 2 * nbuf]
        stg = rest[5 + 2 * nbuf:7 + 2 * nbuf]
        c = lax.axis_index("c")
        s = lax.axis_index("s")
        w = s * NC + c
        sl = pl.ds(s * rpt, rpt)

        def stage(g, pg, sem):
            pltpu.async_copy(srcp.at[w, pl.ds(g * GS, GS)], idxs[pg], sem)
            pltpu.async_copy(dstp.at[w, pl.ds(g * GS, GS)], idxd[pg], sem)

        def stage_wait(g, pg, sem):
            pltpu.make_async_copy(
                srcp.at[w, pl.ds(g * GS, GS)], idxs[pg], sem).wait()
            pltpu.make_async_copy(
                dstp.at[w, pl.ds(g * GS, GS)], idxd[pg], sem).wait()

        pltpu.sync_copy(z_d.at[pl.ds(0, rpt)], acc.at[sl])
        stage(0, 0, stg[0])
        stage_wait(0, 0, stg[0])
        plsc.subcore_barrier()

        for j in range(g_depth):
            pltpu.async_copy(table.at[idxs[0].at[j]], bufs[j], gsem[j])

        def body(j, carry):
            @pl.when(lax.rem(j, GS) == 0)
            def _stage_next():
                g = lax.div(j, GS)
                for pg in range(2):
                    @pl.when(lax.rem(g + 1, 2) == pg)
                    def _s():
                        @pl.when((g + 1) * GS < n_chunks)
                        def _go():
                            stage(g + 1, pg, stg[pg])

            @pl.when(lax.rem(j, GS) == GS - g_depth)
            def _wait_next():
                g = lax.div(j, GS)
                for pg in range(2):
                    @pl.when(lax.rem(g + 1, 2) == pg)
                    def _w():
                        @pl.when((g + 1) * GS < n_chunks)
                        def _done():
                            stage_wait(g + 1, pg, stg[pg])

            nxt = j + g_depth

            @pl.when(nxt < n_chunks)
            def _refill():
                rn = lax.rem(nxt, GS)
                for pg in range(2):
                    @pl.when(lax.rem(lax.div(nxt, GS), 2) == pg)
                    def _g():
                        for par in range(nbuf):
                            @pl.when(lax.rem(nxt, nbuf) == par)
                            def _start():
                                pltpu.async_copy(
                                    table.at[idxs[pg].at[rn]],
                                    bufs[par], gsem[par])

            rj = lax.rem(j, GS)
            for pg in range(2):
                @pl.when(lax.rem(lax.div(j, GS), 2) == pg)
                def _g2():
                    for par in range(nbuf):
                        @pl.when(lax.rem(j, nbuf) == par)
                        def _consume():
                            pltpu.make_async_copy(
                                table.at[idxs[pg].at[rj]],
                                bufs[par], gsem[par]).wait()
                            pltpu.sync_copy(
                                bufs[par], acc.at[idxd[pg].at[rj]], add=True)
            return carry

        lax.fori_loop(0, n_chunks, body, 0)
        plsc.subcore_barrier()
        pltpu.sync_copy(acc.at[sl], out_sum.at[c, sl])

        if with_deg:
            pltpu.sync_copy(z_d.at[pl.ds(0, rpt)], acc.at[sl])
            pltpu.sync_copy(ones_hbm, bufs[0])
            plsc.subcore_barrier()

            def dbody(j, carry):
                @pl.when(lax.rem(j, GS) == 0)
                def _stage():
                    pltpu.sync_copy(
                        dstp.at[w, pl.ds(lax.div(j, GS) * GS, GS)], idxd[0])
                pltpu.sync_copy(bufs[0], acc.at[idxd[0].at[lax.rem(j, GS)]],
                                add=True)
                return carry

            lax.fori_loop(0, n_chunks, dbody, 0)
            plsc.subcore_barrier()
            pltpu.sync_copy(acc.at[sl], out_deg.at[c, sl])

    return seg


def _sage_block(x_ref, xsum_ref, degf_ref, w1l_ref, w1r_ref, b1_ref,
                w2l_ref, w2r_ref, pq_ref):
    deg = jnp.maximum(degf_ref[0] + degf_ref[1], 1.0)
    agg = (xsum_ref[0] + xsum_ref[1]) / deg
    h = (jnp.dot(agg, w1l_ref[...], preferred_element_type=jnp.float32)
         + jnp.dot(x_ref[...], w1r_ref[...], preferred_element_type=jnp.float32)
         + b1_ref[...])
    h = jnp.maximum(h, 0.0)
    p = jnp.dot(h, w2l_ref[...], preferred_element_type=jnp.float32)
    q = jnp.dot(h, w2r_ref[...], preferred_element_type=jnp.float32)
    pq_ref[...] = jnp.concatenate([p, q], axis=1)


def _out_block(psum_ref, pq_ref, degf_ref, b2_ref, o_ref):
    d_out = o_ref.shape[1]
    deg = jnp.maximum(degf_ref[0] + degf_ref[1], 1.0)
    ps = (psum_ref[0] + psum_ref[1]) / deg
    z = ps[:, :d_out] + pq_ref[...][:, d_out:] + b2_ref[...]
    m = jnp.max(z, axis=1, keepdims=True)
    e = z - m
    o_ref[...] = e - jnp.log(jnp.sum(jnp.exp(e), axis=1, keepdims=True))


def kernel(x, W1l, W1r, b1, W2l, W2r, b2, edge_index):
    n, d_in = x.shape
    d_h = W1l.shape[1]
    d_out = W2l.shape[1]
    e = edge_index.shape[1]
    n_pad = -(-(n + 1) // 128) * 128

    nch = -(-e // (NW * CHUNK))
    nch += (-nch) % GS
    e_pad = NW * nch * CHUNK
    srcp = jnp.concatenate(
        [edge_index[0], jnp.zeros((e_pad - e,), jnp.int32)]
    ).reshape(NW, nch, CHUNK)
    dstp = jnp.concatenate(
        [edge_index[1], jnp.full((e_pad - e,), n, jnp.int32)]
    ).reshape(NW, nch, CHUNK)

    z128 = jnp.zeros((n_pad // NS, d_in), jnp.float32)
    ones = jnp.ones((CHUNK, d_in), jnp.float32)

    agg1 = _make_seg_sum(n_pad, d_in, nch, with_deg=True)
    xsum, degf = agg1(x, srcp, dstp, z128, ones)

    rb = 400
    grid = (n // rb,)
    pq = pl.pallas_call(
        _sage_block,
        grid=grid,
        in_specs=[
            pl.BlockSpec((rb, d_in), lambda i: (i, 0)),
            pl.BlockSpec((NC, rb, d_in), lambda i: (0, i, 0)),
            pl.BlockSpec((NC, rb, d_in), lambda i: (0, i, 0)),
            pl.BlockSpec((d_in, d_h), lambda i: (0, 0)),
            pl.BlockSpec((d_in, d_h), lambda i: (0, 0)),
            pl.BlockSpec((1, d_h), lambda i: (0, 0)),
            pl.BlockSpec((d_h, d_out), lambda i: (0, 0)),
            pl.BlockSpec((d_h, d_out), lambda i: (0, 0)),
        ],
        out_specs=pl.BlockSpec((rb, 2 * d_out), lambda i: (i, 0)),
        out_shape=jax.ShapeDtypeStruct((n, 2 * d_out), jnp.float32),
    )(x, xsum, degf, W1l, W1r, b1.reshape(1, d_h), W2l, W2r)

    seg2 = _make_seg_sum(n_pad, 2 * d_out, nch, with_deg=False)
    (psum,) = seg2(pq, srcp, dstp, z128)

    out = pl.pallas_call(
        _out_block,
        grid=grid,
        in_specs=[
            pl.BlockSpec((NC, rb, 2 * d_out), lambda i: (0, i, 0)),
            pl.BlockSpec((rb, 2 * d_out), lambda i: (i, 0)),
            pl.BlockSpec((NC, rb, d_in), lambda i: (0, i, 0)),
            pl.BlockSpec((1, d_out), lambda i: (0, 0)),
        ],
        out_specs=pl.BlockSpec((rb, d_out), lambda i: (i, 0)),
        out_shape=jax.ShapeDtypeStruct((n, d_out), jnp.float32),
    )(psum, pq, degf, b2.reshape(1, d_out))
    return out


# speedup vs baseline: 1.0058x; 1.0058x over previous
import functools

import jax
import jax.numpy as jnp
from jax import lax
from jax.experimental import pallas as pl
from jax.experimental.pallas import tpu as pltpu
from jax.experimental.pallas import tpu_sc as plsc

NC = 2
NS = 16
NW = NC * NS
CHUNK = 64


GS = 16


def _make_seg_sum(n_pad, d, n_chunks, with_deg):
    mesh = plsc.VectorSubcoreMesh(core_axis_name="c", subcore_axis_name="s")
    rpt = n_pad // NS
    nbuf = 5
    g_depth = nbuf - 2
    n_grp = n_chunks // GS

    out_type = [jax.ShapeDtypeStruct((NC, n_pad, d), jnp.float32)]
    if with_deg:
        out_type.append(jax.ShapeDtypeStruct((NC, n_pad, d), jnp.float32))

    scratch = [pltpu.VMEM((GS, CHUNK), jnp.int32)] * 4
    scratch += [pltpu.VMEM((CHUNK, d), jnp.float32)] * nbuf
    scratch.append(pltpu.VMEM_SHARED((n_pad, d), jnp.float32))
    scratch += [pltpu.SemaphoreType.DMA] * (2 * nbuf + 3)

    @functools.partial(
        pl.kernel, mesh=mesh, out_type=tuple(out_type),
        scratch_types=tuple(scratch),
    )
    def seg(*refs):
        if with_deg:
            (table, srcp, dstp, z_d, ones_hbm, out_sum, out_deg,
             *rest) = refs
        else:
            (table, srcp, dstp, z_d, out_sum, *rest) = refs
        idxs = rest[0:2]
        idxd = rest[2:4]
        bufs = rest[4:4 + nbuf]
        acc = rest[4 + nbuf]
        gsem = rest[5 + nbuf:5 + 2 * nbuf]
        ssem = rest[5 + 2 * nbuf:5 + 3 * nbuf]
        stg = rest[5 + 3 * nbuf:7 + 3 * nbuf]
        dsem = rest[7 + 3 * nbuf]
        c = lax.axis_index("c")
        s = lax.axis_index("s")
        w = s * NC + c
        sl = pl.ds(s * rpt, rpt)

        def stage(g, pg, sem):
            pltpu.async_copy(srcp.at[w, pl.ds(g * GS, GS)], idxs[pg], sem)
            pltpu.async_copy(dstp.at[w, pl.ds(g * GS, GS)], idxd[pg], sem)

        def stage_wait(g, pg, sem):
            pltpu.make_async_copy(
                srcp.at[w, pl.ds(g * GS, GS)], idxs[pg], sem).wait()
            pltpu.make_async_copy(
                dstp.at[w, pl.ds(g * GS, GS)], idxd[pg], sem).wait()

        pltpu.sync_copy(z_d.at[pl.ds(0, rpt)], acc.at[sl])
        stage(0, 0, stg[0])
        stage_wait(0, 0, stg[0])
        plsc.subcore_barrier()

        for j in range(g_depth):
            pltpu.async_copy(table.at[idxs[0].at[j]], bufs[j], gsem[j])

        def body(j, carry):
            old = j + g_depth - nbuf

            @pl.when(old >= 0)
            def _retire():
                ro = lax.rem(old, GS)
                for pg in range(2):
                    @pl.when(lax.rem(lax.div(old, GS), 2) == pg)
                    def _p():
                        for par in range(nbuf):
                            @pl.when(lax.rem(old, nbuf) == par)
                            def _w():
                                pltpu.make_async_copy(
                                    bufs[par], acc.at[idxd[pg].at[ro]],
                                    ssem[par]).wait()

            @pl.when(lax.rem(j, GS) == 1)
            def _stage_next():
                g = lax.div(j, GS)
                for pg in range(2):
                    @pl.when(lax.rem(g + 1, 2) == pg)
                    def _s():
                        @pl.when((g + 1) * GS < n_chunks)
                        def _go():
                            stage(g + 1, pg, stg[pg])

            @pl.when(lax.rem(j, GS) == GS - g_depth)
            def _wait_next():
                g = lax.div(j, GS)
                for pg in range(2):
                    @pl.when(lax.rem(g + 1, 2) == pg)
                    def _w():
                        @pl.when((g + 1) * GS < n_chunks)
                        def _done():
                            stage_wait(g + 1, pg, stg[pg])

            nxt = j + g_depth

            @pl.when(nxt < n_chunks)
            def _refill():
                rn = lax.rem(nxt, GS)
                for pg in range(2):
                    @pl.when(lax.rem(lax.div(nxt, GS), 2) == pg)
                    def _g():
                        for par in range(nbuf):
                            @pl.when(lax.rem(nxt, nbuf) == par)
                            def _start():
                                pltpu.async_copy(
                                    table.at[idxs[pg].at[rn]],
                                    bufs[par], gsem[par])

            rj = lax.rem(j, GS)
            for pg in range(2):
                @pl.when(lax.rem(lax.div(j, GS), 2) == pg)
                def _g2():
                    for par in range(nbuf):
                        @pl.when(lax.rem(j, nbuf) == par)
                        def _consume():
                            pltpu.make_async_copy(
                                table.at[idxs[pg].at[rj]],
                                bufs[par], gsem[par]).wait()
                            pltpu.async_copy(
                                bufs[par], acc.at[idxd[pg].at[rj]],
                                ssem[par], add=True)
            return carry

        lax.fori_loop(0, n_chunks, body, 0)
        for t in range(max(0, n_chunks - 2), n_chunks):
            pltpu.make_async_copy(
                bufs[t % nbuf],
                acc.at[idxd[(t // GS) % 2].at[t % GS]],
                ssem[t % nbuf]).wait()
        plsc.subcore_barrier()
        pltpu.sync_copy(acc.at[sl], out_sum.at[c, sl])

        if with_deg:
            pltpu.sync_copy(z_d.at[pl.ds(0, rpt)], acc.at[sl])
            pltpu.sync_copy(ones_hbm, bufs[0])
            plsc.subcore_barrier()

            def dbody(j, carry):
                rj = lax.rem(j, GS)
                for pg in range(2):
                    @pl.when(lax.rem(lax.div(j, GS), 2) == pg)
                    def _g():
                        @pl.when(rj == 0)
                        def _stage():
                            pltpu.sync_copy(
                                dstp.at[w, pl.ds(lax.div(j, GS) * GS, GS)],
                                idxd[pg])

                        @pl.when(j >= 6)
                        def _ret():
                            pltpu.make_async_copy(
                                bufs[0], acc.at[idxd[pg].at[rj]],
                                dsem).wait()
                        pltpu.async_copy(
                            bufs[0], acc.at[idxd[pg].at[rj]], dsem, add=True)
                return carry

            lax.fori_loop(0, n_chunks, dbody, 0)
            for t in range(min(6, n_chunks)):
                pltpu.make_async_copy(
                    bufs[0], acc.at[idxd[0].at[t]], dsem).wait()
            plsc.subcore_barrier()
            pltpu.sync_copy(acc.at[sl], out_deg.at[c, sl])

    return seg


def _sage_block(x_ref, xsum_ref, degf_ref, w1l_ref, w1r_ref, b1_ref,
                w2l_ref, w2r_ref, pq_ref):
    deg = jnp.maximum(degf_ref[0] + degf_ref[1], 1.0)
    agg = (xsum_ref[0] + xsum_ref[1]) / deg
    h = (jnp.dot(agg, w1l_ref[...], preferred_element_type=jnp.float32)
         + jnp.dot(x_ref[...], w1r_ref[...], preferred_element_type=jnp.float32)
         + b1_ref[...])
    h = jnp.maximum(h, 0.0)
    p = jnp.dot(h, w2l_ref[...], preferred_element_type=jnp.float32)
    q = jnp.dot(h, w2r_ref[...], preferred_element_type=jnp.float32)
    pq_ref[...] = jnp.concatenate([p, q], axis=1)


def _out_block(psum_ref, pq_ref, degf_ref, b2_ref, o_ref):
    d_out = o_ref.shape[1]
    deg = jnp.maximum(degf_ref[0] + degf_ref[1], 1.0)
    ps = (psum_ref[0] + psum_ref[1]) / deg
    z = ps[:, :d_out] + pq_ref[...][:, d_out:] + b2_ref[...]
    m = jnp.max(z, axis=1, keepdims=True)
    e = z - m
    o_ref[...] = e - jnp.log(jnp.sum(jnp.exp(e), axis=1, keepdims=True))


def kernel(x, W1l, W1r, b1, W2l, W2r, b2, edge_index):
    n, d_in = x.shape
    d_h = W1l.shape[1]
    d_out = W2l.shape[1]
    e = edge_index.shape[1]
    n_pad = -(-(n + 1) // 128) * 128

    nch = -(-e // (NW * CHUNK))
    nch += (-nch) % GS
    e_pad = NW * nch * CHUNK
    srcp = jnp.concatenate(
        [edge_index[0], jnp.zeros((e_pad - e,), jnp.int32)]
    ).reshape(NW, nch, CHUNK)
    dstp = jnp.concatenate(
        [edge_index[1], jnp.full((e_pad - e,), n, jnp.int32)]
    ).reshape(NW, nch, CHUNK)

    z128 = jnp.zeros((n_pad // NS, d_in), jnp.float32)
    ones = jnp.ones((CHUNK, d_in), jnp.float32)

    agg1 = _make_seg_sum(n_pad, d_in, nch, with_deg=True)
    xsum, degf = agg1(x, srcp, dstp, z128, ones)

    rb = 400
    grid = (n // rb,)
    pq = pl.pallas_call(
        _sage_block,
        grid=grid,
        in_specs=[
            pl.BlockSpec((rb, d_in), lambda i: (i, 0)),
            pl.BlockSpec((NC, rb, d_in), lambda i: (0, i, 0)),
            pl.BlockSpec((NC, rb, d_in), lambda i: (0, i, 0)),
            pl.BlockSpec((d_in, d_h), lambda i: (0, 0)),
            pl.BlockSpec((d_in, d_h), lambda i: (0, 0)),
            pl.BlockSpec((1, d_h), lambda i: (0, 0)),
            pl.BlockSpec((d_h, d_out), lambda i: (0, 0)),
            pl.BlockSpec((d_h, d_out), lambda i: (0, 0)),
        ],
        out_specs=pl.BlockSpec((rb, 2 * d_out), lambda i: (i, 0)),
        out_shape=jax.ShapeDtypeStruct((n, 2 * d_out), jnp.float32),
    )(x, xsum, degf, W1l, W1r, b1.reshape(1, d_h), W2l, W2r)

    seg2 = _make_seg_sum(n_pad, 2 * d_out, nch, with_deg=False)
    (psum,) = seg2(pq, srcp, dstp, z128)

    out = pl.pallas_call(
        _out_block,
        grid=grid,
        in_specs=[
            pl.BlockSpec((NC, rb, 2 * d_out), lambda i: (0, i, 0)),
            pl.BlockSpec((rb, 2 * d_out), lambda i: (i, 0)),
            pl.BlockSpec((NC, rb, d_in), lambda i: (0, i, 0)),
            pl.BlockSpec((1, d_out), lambda i: (0, 0)),
        ],
        out_specs=pl.BlockSpec((rb, d_out), lambda i: (i, 0)),
        out_shape=jax.ShapeDtypeStruct((n, d_out), jnp.float32),
    )(psum, pq, degf, b2.reshape(1, d_out))
    return out
